# Initial kernel scaffold; baseline (speedup 1.0000x reference)
import jax
import jax.numpy as jnp
from jax.experimental import pallas as pl


def kernel(x, edge_index, Wl0, Wr0, b0, g0, be0, rm0, rv0, Wl1, Wr1, b1, g1, be1, rm1, rv1, Wl2, Wr2, b2, g2, be2, rm2, rv2, Wc, bc):
    raise NotImplementedError("write your pallas kernel here")


# speedup vs baseline: 4.0568x; 4.0568x over previous
import jax
import jax.numpy as jnp
from jax import lax
from jax.experimental import pallas as pl
from jax.experimental.pallas import tpu as pltpu
from jax.experimental.pallas import tpu_sc as plsc

_N = 10000
_E = 320000
_NT = 16
_NR = 632
_NPAD = _NT * _NR
_K = 128
_C12 = 157
_C0 = 79
_BN = 1000
_GRID = _N // _BN

_MESH = plsc.VectorSubcoreMesh(
    core_axis_name="c", subcore_axis_name="s", num_cores=2)


def _agg0_body(x_hbm, src3, dst3, zrows, ones_hbm, agg_out, deg_out,
               src_v, dst_v, data_v, ones_v, acc, sem):
  c = lax.axis_index("c")
  s = lax.axis_index("s")
  w = c * _NT + s
  r0 = s * _NR

  pltpu.sync_copy(ones_hbm, ones_v)

  pltpu.sync_copy(zrows, acc.at[pl.ds(r0, _NR)])
  plsc.subcore_barrier()

  def step_deg(j, carry):
    pltpu.sync_copy(dst3.at[w, j], dst_v)
    pltpu.sync_copy(ones_v, acc.at[dst_v], add=True)
    return carry

  lax.fori_loop(0, _C0, step_deg, 0)
  plsc.subcore_barrier()
  pltpu.sync_copy(acc.at[pl.ds(r0, _NR)],
                  deg_out.at[pl.ds(c * _NPAD + r0, _NR)])

  pltpu.sync_copy(zrows, acc.at[pl.ds(r0, _NR)])
  plsc.subcore_barrier()

  def step_agg(j, carry):
    pltpu.sync_copy(src3.at[w, j], src_v)
    pltpu.sync_copy(dst3.at[w, j], dst_v)
    pltpu.async_copy(x_hbm.at[src_v], data_v, sem).wait()
    pltpu.sync_copy(data_v, acc.at[dst_v], add=True)
    return carry

  lax.fori_loop(0, _C0, step_agg, 0)
  plsc.subcore_barrier()
  pltpu.sync_copy(acc.at[pl.ds(r0, _NR)],
                  agg_out.at[pl.ds(c * _NPAD + r0, _NR)])


_AGG0 = pl.kernel(
    _agg0_body,
    mesh=_MESH,
    out_type=[jax.ShapeDtypeStruct((2 * _NPAD, 128), jnp.float32),
              jax.ShapeDtypeStruct((2 * _NPAD, 128), jnp.float32)],
    scratch_types=[
        pltpu.VMEM((_K,), jnp.int32),
        pltpu.VMEM((_K,), jnp.int32),
        pltpu.VMEM((_K, 128), jnp.float32),
        pltpu.VMEM((_K, 128), jnp.float32),
        pltpu.VMEM_SHARED((_NPAD, 128), jnp.float32),
        pltpu.SemaphoreType.DMA,
    ],
)


def _agg12_body(h2, src3, dst3, zrows, out,
                src_v, idx_v, dst_v, data_v, acc, sem):
  c = lax.axis_index("c")
  s = lax.axis_index("s")
  r0 = s * _NR
  cn = c * _N

  pltpu.sync_copy(zrows, acc.at[pl.ds(r0, _NR)])
  plsc.subcore_barrier()

  def step(j, carry):
    pltpu.sync_copy(src3.at[s, j], src_v)
    pltpu.sync_copy(dst3.at[s, j], dst_v)
    for k in range(_K // 16):
      sl = pl.ds(k * 16, 16)
      idx_v[sl] = src_v[sl] + cn
    pltpu.async_copy(h2.at[idx_v], data_v, sem).wait()
    pltpu.sync_copy(data_v, acc.at[dst_v], add=True)
    return carry

  lax.fori_loop(0, _C12, step, 0)
  plsc.subcore_barrier()
  pltpu.sync_copy(acc.at[pl.ds(r0, _NR)],
                  out.at[pl.ds(c * _NPAD + r0, _NR)])


_AGG12 = pl.kernel(
    _agg12_body,
    mesh=_MESH,
    out_type=jax.ShapeDtypeStruct((2 * _NPAD, 128), jnp.float32),
    scratch_types=[
        pltpu.VMEM((_K,), jnp.int32),
        pltpu.VMEM((_K,), jnp.int32),
        pltpu.VMEM((_K,), jnp.int32),
        pltpu.VMEM((_K, 128), jnp.float32),
        pltpu.VMEM_SHARED((_NPAD, 128), jnp.float32),
        pltpu.SemaphoreType.DMA,
    ],
)


def _bn_relu(z, bn):
  scale = bn[1] * lax.rsqrt(bn[4] + 1e-5)
  shift = (bn[0] - bn[3]) * scale + bn[2]
  return jnp.maximum(z * scale[None, :] + shift[None, :], 0.0)


def _recip_deg(deg):
  return 1.0 / jnp.maximum(deg[0][:, 0:1] + deg[1][:, 0:1], 1.0)


def _dot(a, b):
  return jnp.dot(a, b, preferred_element_type=jnp.float32)


def _dense0_body(agg_ref, deg_ref, x_ref, wl_ref, wr_ref, bn_ref, out_ref):
  agg = agg_ref[...]
  a = (agg[0] + agg[1]) * _recip_deg(deg_ref[...])
  z = _dot(a, wl_ref[...]) + _dot(x_ref[...], wr_ref[...])
  y = _bn_relu(z, bn_ref[...])
  out_ref[0] = y[:, :128]
  out_ref[1] = y[:, 128:]


def _dense1_body(agg_ref, deg_ref, h_ref, wl_ref, wr_ref, bn_ref, out_ref):
  agg = agg_ref[...]
  recip = _recip_deg(deg_ref[...])
  wl = wl_ref[...]
  wr = wr_ref[...]
  h = h_ref[...]
  z = (_dot(agg[0] * recip, wl[:128]) + _dot(agg[1] * recip, wl[128:])
       + _dot(h[0], wr[:128]) + _dot(h[1], wr[128:]))
  y = _bn_relu(z, bn_ref[...])
  out_ref[0] = y[:, :128]
  out_ref[1] = y[:, 128:]


def _dense2_body(agg_ref, deg_ref, h_ref, wl_ref, wr_ref, bn_ref,
                 wc_ref, bc_ref, out_ref):
  agg = agg_ref[...]
  recip = _recip_deg(deg_ref[...])
  wl = wl_ref[...]
  wr = wr_ref[...]
  h = h_ref[...]
  z = (_dot(agg[0] * recip, wl[:128]) + _dot(agg[1] * recip, wl[128:])
       + _dot(h[0], wr[:128]) + _dot(h[1], wr[128:]))
  y = _bn_relu(z, bn_ref[...])
  out_ref[...] = _dot(y, wc_ref[...]) + bc_ref[...]


def _full(shape):
  return pl.BlockSpec(shape, lambda i: tuple(0 for _ in shape))


_AGG_SPEC = pl.BlockSpec((2, _BN, 128), lambda i: (0, i, 0))

_DENSE0 = pl.pallas_call(
    _dense0_body,
    grid=(_GRID,),
    in_specs=[
        _AGG_SPEC,
        _AGG_SPEC,
        pl.BlockSpec((_BN, 128), lambda i: (i, 0)),
        _full((128, 256)),
        _full((128, 256)),
        _full((5, 256)),
    ],
    out_specs=pl.BlockSpec((2, _BN, 128), lambda i: (0, i, 0)),
    out_shape=jax.ShapeDtypeStruct((2, _N, 128), jnp.float32),
)

_DENSE1 = pl.pallas_call(
    _dense1_body,
    grid=(_GRID,),
    in_specs=[
        _AGG_SPEC,
        _AGG_SPEC,
        pl.BlockSpec((2, _BN, 128), lambda i: (0, i, 0)),
        _full((256, 256)),
        _full((256, 256)),
        _full((5, 256)),
    ],
    out_specs=pl.BlockSpec((2, _BN, 128), lambda i: (0, i, 0)),
    out_shape=jax.ShapeDtypeStruct((2, _N, 128), jnp.float32),
)

_DENSE2 = pl.pallas_call(
    _dense2_body,
    grid=(_GRID,),
    in_specs=[
        _AGG_SPEC,
        _AGG_SPEC,
        pl.BlockSpec((2, _BN, 128), lambda i: (0, i, 0)),
        _full((256, 128)),
        _full((256, 128)),
        _full((5, 128)),
        _full((128, 2)),
        _full((1, 2)),
    ],
    out_specs=pl.BlockSpec((_BN, 2), lambda i: (i, 0)),
    out_shape=jax.ShapeDtypeStruct((_N, 2), jnp.float32),
)


def kernel(x, edge_index,
           Wl0, Wr0, b0, g0, be0, rm0, rv0,
           Wl1, Wr1, b1, g1, be1, rm1, rv1,
           Wl2, Wr2, b2, g2, be2, rm2, rv2,
           Wc, bc):
  src = edge_index[0]
  dst = edge_index[1]

  e0 = 2 * _NT * _C0 * _K
  src0 = jnp.concatenate(
      [src, jnp.zeros((e0 - _E,), jnp.int32)]).reshape(2 * _NT, _C0, _K)
  dst0 = jnp.concatenate(
      [dst, jnp.full((e0 - _E,), _N, jnp.int32)]).reshape(2 * _NT, _C0, _K)

  e12 = _NT * _C12 * _K
  src12 = jnp.concatenate(
      [src, jnp.zeros((e12 - _E,), jnp.int32)]).reshape(_NT, _C12, _K)
  dst12 = jnp.concatenate(
      [dst, jnp.full((e12 - _E,), _N, jnp.int32)]).reshape(_NT, _C12, _K)

  zrows = jnp.zeros((_NR, 128), jnp.float32)
  ones = jnp.ones((_K, 128), jnp.float32)

  bn0 = jnp.stack([b0, g0, be0, rm0, rv0])
  bn1 = jnp.stack([b1, g1, be1, rm1, rv1])
  bn2 = jnp.stack([b2, g2, be2, rm2, rv2])

  agg0, deg = _AGG0(x, src0, dst0, zrows, ones)
  deg = deg.reshape(2, _NPAD, 128)
  h0 = _DENSE0(agg0.reshape(2, _NPAD, 128), deg, x, Wl0, Wr0, bn0)

  agg1 = _AGG12(h0.reshape(2 * _N, 128), src12, dst12, zrows)
  h1 = _DENSE1(agg1.reshape(2, _NPAD, 128), deg, h0, Wl1, Wr1, bn1)

  agg2 = _AGG12(h1.reshape(2 * _N, 128), src12, dst12, zrows)
  return _DENSE2(agg2.reshape(2, _NPAD, 128), deg, h1, Wl2, Wr2, bn2,
                 Wc, bc.reshape(1, 2))
